# baseline (speedup 1.0000x reference)
import functools

import jax
import jax.numpy as jnp
from jax import lax
from jax.experimental import pallas as pl
from jax.experimental.pallas import tpu as pltpu
from jax.experimental.pallas import tpu_sc as plsc

N_NODES_C = 100000
N_EDGES_C = 3200000
D = 16

NC = 2
NS = 16
NW = NC * NS

U = 128
CH = 2
NU = N_EDGES_C // U
BASE_CNT = (NU // NW) & ~7
EXTRA_TILES = (NU - NW * BASE_CNT) // 8

N_PAD = 100096
RPT = N_PAD // NS

NSLOT = 2
NIDX = 4
UNROLL = 4

_mesh = plsc.VectorSubcoreMesh(
    core_axis_name="c", subcore_axis_name="s", num_cores=NC, num_subcores=NS
)


@functools.partial(
    pl.kernel,
    mesh=_mesh,
    out_type=jax.ShapeDtypeStruct((NC, N_PAD, D), jnp.float32),
    scratch_types=[
        pltpu.VMEM((NSLOT, 2, CH, 8, U), jnp.float32),
        pltpu.VMEM((NSLOT, CH * U, D), jnp.float32),
        pltpu.VMEM((NIDX, CH * U), jnp.int32),
        pltpu.VMEM_SHARED((N_PAD, D), jnp.float32),
        [pltpu.SemaphoreType.DMA] * NSLOT,
        [pltpu.SemaphoreType.DMA] * NSLOT,
        [pltpu.SemaphoreType.DMA] * NSLOT,
    ],
    compiler_params=pltpu.CompilerParams(use_tc_tiling_on_sc=False,
                                         needs_layout_passes=False),
)
def _sc_scatter_add(feat_hbm, eidx_hbm, out_hbm, raw_v, rows_v, idx_v,
                    acc, sem_f, sem_i, sem_s):
    cid = lax.axis_index("c")
    sid = lax.axis_index("s")
    wid = cid * NS + sid
    cnt = jnp.where(wid < EXTRA_TILES, BASE_CNT + 8, BASE_CNT)
    base_u = wid * BASE_CNT + 8 * jnp.minimum(wid, EXTRA_TILES)
    nch = cnt // CH

    zs = jnp.minimum(cid, 0)
    lane = jnp.arange(16, dtype=jnp.int32) + zs
    i_dt = lane // 8
    i_di = lane % 8
    vb0 = jnp.full((16,), 0, jnp.int32) + zs

    zero_row = jnp.zeros((D,), jnp.float32)

    @pl.loop(0, CH * U)
    def _(i):
        rows_v[0, i, :] = zero_row

    r0 = sid * RPT
    ZR = CH * U
    for k in range(RPT // ZR):
        pltpu.sync_copy(rows_v.at[0], acc.at[pl.ds(r0 + k * ZR, ZR)])
    rem = RPT % ZR
    if rem:
        pltpu.sync_copy(rows_v.at[0, pl.ds(0, rem)],
                        acc.at[pl.ds(r0 + (RPT // ZR) * ZR, rem)])

    def issue_fetch(j, fslot, islot):
        u_abs = base_u + j * CH
        pltpu.async_copy(feat_hbm.at[pl.ds(0, 2), pl.ds(u_abs, CH)],
                         raw_v.at[fslot], sem_f[fslot])
        for u in range(CH):
            pltpu.async_copy(
                eidx_hbm.at[pl.ds((2 * (u_abs + u) + 1) * U, U)],
                idx_v.at[islot, pl.ds(u * U, U)], sem_i[fslot])

    def wait_fetch(fslot, islot):
        pltpu.make_async_copy(feat_hbm.at[pl.ds(0, 2), pl.ds(0, CH)],
                              raw_v.at[fslot], sem_f[fslot]).wait()
        for u in range(CH):
            pltpu.make_async_copy(eidx_hbm.at[pl.ds(0, U)],
                                  idx_v.at[islot, pl.ds(u * U, U)],
                                  sem_i[fslot]).wait()

    def wait_scatter(fslot):
        pltpu.make_async_copy(rows_v.at[fslot],
                              acc.at[idx_v.at[fslot]],
                              sem_s[fslot]).wait()

    for k in range(NSLOT):
        issue_fetch(k, k, k)

    plsc.subcore_barrier()

    def visit(j, f, islot):
        @pl.when(j >= NSLOT)
        def _():
            wait_scatter(f)

        wait_fetch(f, islot)

        for u in range(CH):
            vb_u = vb0 + u

            @pl.loop(0, U, step=8)
            def _(e0):
                vals = [plsc.load_gather(raw_v.at[f],
                                         [i_dt, vb_u, i_di, vb0 + (e0 + k)])
                        for k in range(8)]
                for k in range(8):
                    rows_v[f, u * U + e0 + k, :] = vals[k]

        pltpu.async_copy(rows_v.at[f], acc.at[idx_v.at[islot]],
                         sem_s[f], add=True)

        @pl.when(j + NSLOT < nch)
        def _():
            issue_fetch(j + NSLOT, f, (islot + NSLOT) % NIDX)

    nch_main = (nch // UNROLL) * UNROLL

    @pl.loop(0, nch_main, step=UNROLL)
    def _(jj):
        for s in range(UNROLL):
            visit(jj + s, s % NSLOT, s)

    for s in range(UNROLL - 1):
        @pl.when(nch_main + s < nch)
        def _():
            visit(nch_main + s, s % NSLOT, s)

    for f in range(NSLOT):
        wait_scatter(f)

    plsc.subcore_barrier()
    pltpu.sync_copy(acc.at[pl.ds(r0, RPT)], out_hbm.at[cid, pl.ds(r0, RPT)])


ROWS_IN = N_PAD * D // 128
ROWS_OUT = N_NODES_C * D // 128
RBLK = 544
GRID = -(-ROWS_OUT // RBLK)


def _combine_body(nn_ref, p_ref, o_ref):
    i = pl.program_id(0)
    s = p_ref[0] + p_ref[1]
    r = lax.broadcasted_iota(jnp.int32, (RBLK, 128), 0) + i * RBLK
    c = lax.broadcasted_iota(jnp.int32, (RBLK, 128), 1)
    node = r * 8 + c // D
    o_ref[...] = jnp.where(node < nn_ref[0, 0], s, 0.0)


_combine = pl.pallas_call(
    _combine_body,
    grid=(GRID,),
    in_specs=[
        pl.BlockSpec(memory_space=pltpu.SMEM),
        pl.BlockSpec((NC, RBLK, 128), lambda i: (0, i, 0)),
    ],
    out_specs=pl.BlockSpec((RBLK, 128), lambda i: (i, 0)),
    out_shape=jax.ShapeDtypeStruct((ROWS_OUT, 128), jnp.float32),
)


def kernel(edge_feat, edge_index, num_nodes):
    x4 = edge_feat.T.reshape(2, 8, NU, U).transpose(0, 2, 1, 3)
    eflat = edge_index.reshape(2, NU, U).transpose(1, 0, 2).reshape(-1)
    parts = _sc_scatter_add(x4, eflat)
    p = parts.reshape(NC, ROWS_IN, 128)
    nn = jnp.asarray(num_nodes, jnp.int32).reshape(1, 1)
    out = _combine(nn, p)
    return out.reshape(N_NODES_C, D)


# speedup vs baseline: 1.4702x; 1.4702x over previous
import functools

import jax
import jax.numpy as jnp
from jax import lax
from jax.experimental import pallas as pl
from jax.experimental.pallas import tpu as pltpu
from jax.experimental.pallas import tpu_sc as plsc

N_NODES_C = 100000
N_EDGES_C = 3200000
D = 16

NC = 2
NS = 16
NW = NC * NS

U = 128
CH = 2
NU = N_EDGES_C // U
BASE_CNT = (NU // NW) & ~7
EXTRA_TILES = (NU - NW * BASE_CNT) // 8

N_PAD = 100096
RPT = N_PAD // NS

NSLOT = 2
NIDX = 4
UNROLL = 4

_mesh = plsc.VectorSubcoreMesh(
    core_axis_name="c", subcore_axis_name="s", num_cores=NC, num_subcores=NS
)


@functools.partial(
    pl.kernel,
    mesh=_mesh,
    out_type=jax.ShapeDtypeStruct((NC, N_PAD, D), jnp.float32),
    scratch_types=[
        pltpu.VMEM((NSLOT, 2, CH, 8, U), jnp.float32),
        pltpu.VMEM((NSLOT, CH * U, D), jnp.float32),
        pltpu.VMEM((NIDX, CH * U), jnp.int32),
        pltpu.VMEM_SHARED((N_PAD, D), jnp.float32),
        [pltpu.SemaphoreType.DMA] * NSLOT,
        [pltpu.SemaphoreType.DMA] * NSLOT,
        [pltpu.SemaphoreType.DMA] * NSLOT,
    ],
    compiler_params=pltpu.CompilerParams(use_tc_tiling_on_sc=False,
                                         needs_layout_passes=False),
)
def _sc_scatter_add(feat_hbm, eidx_hbm, out_hbm, raw_v, rows_v, idx_v,
                    acc, sem_f, sem_i, sem_s):
    cid = lax.axis_index("c")
    sid = lax.axis_index("s")
    wid = cid * NS + sid
    cnt = jnp.where(wid < EXTRA_TILES, BASE_CNT + 8, BASE_CNT)
    base_u = wid * BASE_CNT + 8 * jnp.minimum(wid, EXTRA_TILES)
    nch = cnt // CH

    zero_row = jnp.zeros((D,), jnp.float32)

    @pl.loop(0, CH * U)
    def _(i):
        rows_v[0, i, :] = zero_row

    zvec = plsc.bitcast(rows_v[0, 0, :], jnp.int32)
    lane = jnp.arange(16, dtype=jnp.int32) + zvec
    vb0 = zvec

    r0 = sid * RPT
    ZR = CH * U
    for k in range(RPT // ZR):
        pltpu.sync_copy(rows_v.at[0], acc.at[pl.ds(r0 + k * ZR, ZR)])
    rem = RPT % ZR
    if rem:
        pltpu.sync_copy(rows_v.at[0, pl.ds(0, rem)],
                        acc.at[pl.ds(r0 + (RPT // ZR) * ZR, rem)])

    def issue_fetch(j, fslot, islot):
        u_abs = base_u + j * CH
        pltpu.async_copy(feat_hbm.at[pl.ds(0, 2), pl.ds(u_abs, CH)],
                         raw_v.at[fslot], sem_f[fslot])
        for u in range(CH):
            pltpu.async_copy(
                eidx_hbm.at[pl.ds((2 * (u_abs + u) + 1) * U, U)],
                idx_v.at[islot, pl.ds(u * U, U)], sem_i[fslot])

    def wait_fetch(fslot, islot):
        pltpu.make_async_copy(feat_hbm.at[pl.ds(0, 2), pl.ds(0, CH)],
                              raw_v.at[fslot], sem_f[fslot]).wait()
        for u in range(CH):
            pltpu.make_async_copy(eidx_hbm.at[pl.ds(0, U)],
                                  idx_v.at[islot, pl.ds(u * U, U)],
                                  sem_i[fslot]).wait()

    def wait_scatter(fslot):
        pltpu.make_async_copy(rows_v.at[fslot],
                              acc.at[idx_v.at[fslot]],
                              sem_s[fslot]).wait()

    for k in range(NSLOT):
        issue_fetch(k, k, k)

    plsc.subcore_barrier()

    def visit(j, f, islot):
        @pl.when(j >= NSLOT)
        def _():
            wait_scatter(f)

        wait_fetch(f, islot)

        for u in range(CH):
            for g in range(8):
                vals = [raw_v[f, d // 8, u, d % 8, pl.ds(g * 16, 16)]
                        for d in range(D)]
                erow = lane + (u * U + g * 16)
                for d in range(D):
                    plsc.store_scatter(rows_v.at[f], [erow, vb0 + d],
                                       vals[d])

        pltpu.async_copy(rows_v.at[f], acc.at[idx_v.at[islot]],
                         sem_s[f], add=True)

        @pl.when(j + NSLOT < nch)
        def _():
            issue_fetch(j + NSLOT, f, (islot + NSLOT) % NIDX)

    nch_main = (nch // UNROLL) * UNROLL

    @pl.loop(0, nch_main, step=UNROLL)
    def _(jj):
        for s in range(UNROLL):
            visit(jj + s, s % NSLOT, s)

    for s in range(UNROLL - 1):
        @pl.when(nch_main + s < nch)
        def _():
            visit(nch_main + s, s % NSLOT, s)

    for f in range(NSLOT):
        wait_scatter(f)

    plsc.subcore_barrier()
    pltpu.sync_copy(acc.at[pl.ds(r0, RPT)], out_hbm.at[cid, pl.ds(r0, RPT)])


ROWS_IN = N_PAD * D // 128
ROWS_OUT = N_NODES_C * D // 128
RBLK = 544
GRID = -(-ROWS_OUT // RBLK)


def _combine_body(nn_ref, p_ref, o_ref):
    i = pl.program_id(0)
    s = p_ref[0] + p_ref[1]
    r = lax.broadcasted_iota(jnp.int32, (RBLK, 128), 0) + i * RBLK
    c = lax.broadcasted_iota(jnp.int32, (RBLK, 128), 1)
    node = r * 8 + c // D
    o_ref[...] = jnp.where(node < nn_ref[0, 0], s, 0.0)


_combine = pl.pallas_call(
    _combine_body,
    grid=(GRID,),
    in_specs=[
        pl.BlockSpec(memory_space=pltpu.SMEM),
        pl.BlockSpec((NC, RBLK, 128), lambda i: (0, i, 0)),
    ],
    out_specs=pl.BlockSpec((RBLK, 128), lambda i: (i, 0)),
    out_shape=jax.ShapeDtypeStruct((ROWS_OUT, 128), jnp.float32),
)


def kernel(edge_feat, edge_index, num_nodes):
    x4 = edge_feat.T.reshape(2, 8, NU, U).transpose(0, 2, 1, 3)
    eflat = edge_index.reshape(2, NU, U).transpose(1, 0, 2).reshape(-1)
    parts = _sc_scatter_add(x4, eflat)
    p = parts.reshape(NC, ROWS_IN, 128)
    nn = jnp.asarray(num_nodes, jnp.int32).reshape(1, 1)
    out = _combine(nn, p)
    return out.reshape(N_NODES_C, D)
